# baseline (speedup 1.0000x reference)
import functools

import jax
import jax.numpy as jnp
from jax import lax
from jax.experimental import pallas as pl
from jax.experimental.pallas import tpu as pltpu
from jax.experimental.pallas import tpu_sc as plsc

NC, NS, L = 2, 16, 16
NW = NC * NS
TOTAL = 16384 * 1000
PER_W = TOTAL // NW
BLK = 16_000
NBLK = PER_W // BLK
U = 4
VEC_ITERS = BLK // (U * L)

_mesh = plsc.VectorSubcoreMesh(core_axis_name="c", subcore_axis_name="s")


@functools.partial(
    pl.kernel,
    out_type=[
        jax.ShapeDtypeStruct((NW * L,), jnp.float32),
        jax.ShapeDtypeStruct((NW * L,), jnp.float32),
    ],
    mesh=_mesh,
    scratch_types=[
        pltpu.VMEM((2, BLK), jnp.float32),
        pltpu.VMEM((2, BLK), jnp.float32),
        pltpu.VMEM((L,), jnp.float32),
        pltpu.VMEM((L,), jnp.float32),
        pltpu.SemaphoreType.DMA,
        pltpu.SemaphoreType.DMA,
        pltpu.SemaphoreType.DMA,
        pltpu.SemaphoreType.DMA,
    ],
)
def _sc_partials(p_hbm, t_hbm, sums_hbm, cnts_hbm,
                 pbuf, tbuf, svec, cvec, sp0, sp1, st0, st1):
    wid = lax.axis_index("s") * NC + lax.axis_index("c")
    base = wid * PER_W
    sems_p = (sp0, sp1)
    sems_t = (st0, st1)

    def start(i, b):
        pltpu.async_copy(p_hbm.at[pl.ds(base + i * BLK, BLK)], pbuf.at[b],
                         sems_p[b])
        pltpu.async_copy(t_hbm.at[pl.ds(base + i * BLK, BLK)], tbuf.at[b],
                         sems_t[b])

    def wait(b):
        pltpu.make_async_copy(p_hbm.at[pl.ds(base, BLK)], pbuf.at[b],
                              sems_p[b]).wait()
        pltpu.make_async_copy(t_hbm.at[pl.ds(base, BLK)], tbuf.at[b],
                              sems_t[b]).wait()

    def compute(b, accs):
        pb = pbuf.at[b]
        tb = tbuf.at[b]

        def inner(j, c):
            c = list(c)
            for u in range(U):
                off = (j * U + u) * L
                p = pb[pl.ds(off, L)]
                t = tb[pl.ds(off, L)]
                m = t != 0.0
                d = jnp.where(m, p - t, 0.0)
                c[u] = c[u] + d * d
                c[U + u] = c[U + u] + jnp.where(m, 1.0, 0.0)
            return tuple(c)

        return lax.fori_loop(0, VEC_ITERS, inner, accs)

    zero = jnp.zeros((L,), jnp.float32)
    accs = (zero,) * (2 * U)

    start(0, 0)
    start(1, 1)

    def outer(g, accs):
        for b in range(2):
            wait(b)
            accs = compute(b, accs)
            start(2 * g + b + 2, b)
        return accs

    accs = lax.fori_loop(0, NBLK // 2 - 1, outer, accs)
    for b in range(2):
        wait(b)
        accs = compute(b, accs)

    s = accs[0]
    c = accs[U]
    for u in range(1, U):
        s = s + accs[u]
        c = c + accs[U + u]
    svec[...] = s
    cvec[...] = c
    pltpu.sync_copy(svec, sums_hbm.at[pl.ds(wid * L, L)])
    pltpu.sync_copy(cvec, cnts_hbm.at[pl.ds(wid * L, L)])


def _finish_body(sums_ref, cnts_ref, out_ref):
    loss = jnp.sum(sums_ref[...]) / jnp.sum(cnts_ref[...])
    out_ref[...] = loss.reshape(1, 1)


def kernel(predicted, target):
    pf = predicted.reshape(-1)
    tf = target.reshape(-1)
    sums, cnts = _sc_partials(pf, tf)
    loss = pl.pallas_call(
        _finish_body,
        out_shape=jax.ShapeDtypeStruct((1, 1), jnp.float32),
    )(sums.reshape(1, NW * L), cnts.reshape(1, NW * L))
    return loss[0, 0]


# speedup vs baseline: 1.7485x; 1.7485x over previous
import functools

import jax
import jax.numpy as jnp
from jax import lax
from jax.experimental import pallas as pl
from jax.experimental.pallas import tpu as pltpu
from jax.experimental.pallas import tpu_sc as plsc

NC, NS, L = 2, 16, 16
NW = NC * NS
NROWS, NCOLS = 16384, 1000
ROWS_PER_W = NROWS // NW
RB = 16
NBLK = ROWS_PER_W // RB
CHUNKS = NCOLS // L
CU = 2

_mesh = plsc.VectorSubcoreMesh(core_axis_name="c", subcore_axis_name="s")


@functools.partial(
    pl.kernel,
    out_type=[
        jax.ShapeDtypeStruct((1, NW * L), jnp.float32),
        jax.ShapeDtypeStruct((1, NW * L), jnp.float32),
    ],
    mesh=_mesh,
    scratch_types=[
        pltpu.VMEM((2, RB, NCOLS), jnp.float32),
        pltpu.VMEM((2, RB, NCOLS), jnp.float32),
        pltpu.VMEM((L,), jnp.float32),
        pltpu.VMEM((L,), jnp.float32),
        pltpu.SemaphoreType.DMA,
        pltpu.SemaphoreType.DMA,
        pltpu.SemaphoreType.DMA,
        pltpu.SemaphoreType.DMA,
    ],
)
def _sc_partials(p_hbm, t_hbm, sums_hbm, cnts_hbm,
                 pbuf, tbuf, svec, cvec, sp0, sp1, st0, st1):
    wid = lax.axis_index("s") * NC + lax.axis_index("c")
    row0 = wid * ROWS_PER_W
    sems_p = (sp0, sp1)
    sems_t = (st0, st1)
    lane = lax.iota(jnp.int32, L)
    edge_keep = lane >= (L - (NCOLS - CHUNKS * L))

    def start(i, b):
        r = row0 + i * RB
        pltpu.async_copy(p_hbm.at[pl.ds(r, RB), :], pbuf.at[b], sems_p[b])
        pltpu.async_copy(t_hbm.at[pl.ds(r, RB), :], tbuf.at[b], sems_t[b])

    def wait(b):
        pltpu.make_async_copy(p_hbm.at[pl.ds(row0, RB), :], pbuf.at[b],
                              sems_p[b]).wait()
        pltpu.make_async_copy(t_hbm.at[pl.ds(row0, RB), :], tbuf.at[b],
                              sems_t[b]).wait()

    def accum(p, t, s, c):
        m = t != 0.0
        d = jnp.where(m, p - t, 0.0)
        return s + d * d, c + jnp.where(m, 1.0, 0.0)

    def compute(b, accs):
        pb = pbuf.at[b]
        tb = tbuf.at[b]

        def row_body(r, accs):
            def chunk_body(k, a):
                a = list(a)
                for u in range(CU):
                    off = (k * CU + u) * L
                    p = pb[r, pl.ds(off, L)]
                    t = tb[r, pl.ds(off, L)]
                    a[2 * u], a[2 * u + 1] = accum(p, t, a[2 * u], a[2 * u + 1])
                return tuple(a)

            accs = lax.fori_loop(0, CHUNKS // CU, chunk_body, accs)
            off = NCOLS - L
            p = pb[r, pl.ds(off, L)]
            t = jnp.where(edge_keep, tb[r, pl.ds(off, L)], 0.0)
            s0, c0 = accum(p, t, accs[0], accs[1])
            return (s0, c0) + tuple(accs[2:])

        return lax.fori_loop(0, RB, row_body, accs)

    zero = jnp.zeros((L,), jnp.float32)
    accs = (zero,) * (2 * CU)

    start(0, 0)
    start(1, 1)

    def outer(g, accs):
        for b in range(2):
            wait(b)
            accs = compute(b, accs)
            start(2 * g + b + 2, b)
        return accs

    accs = lax.fori_loop(0, NBLK // 2 - 1, outer, accs)
    for b in range(2):
        wait(b)
        accs = compute(b, accs)

    s = accs[0]
    c = accs[1]
    for u in range(1, CU):
        s = s + accs[2 * u]
        c = c + accs[2 * u + 1]
    svec[...] = s
    cvec[...] = c
    pltpu.sync_copy(svec, sums_hbm.at[0, pl.ds(wid * L, L)])
    pltpu.sync_copy(cvec, cnts_hbm.at[0, pl.ds(wid * L, L)])


def _finish_body(sums_ref, cnts_ref, out_ref):
    loss = jnp.sum(sums_ref[...]) / jnp.sum(cnts_ref[...])
    out_ref[...] = loss.reshape(1, 1)


def kernel(predicted, target):
    sums, cnts = _sc_partials(predicted, target)
    loss = pl.pallas_call(
        _finish_body,
        out_shape=jax.ShapeDtypeStruct((1, 1), jnp.float32),
    )(sums, cnts)
    return loss[0, 0]
